# baseline (speedup 1.0000x reference)
import jax
import jax.numpy as jnp
from jax import lax
from jax.experimental import pallas as pl
from jax.experimental.pallas import tpu as pltpu
from jax.experimental.pallas import tpu_sc as plsc

NC, NS = 2, 16
NW = NC * NS
ROWLEN = 128


def _sc_gather(img_flat, xa, ya, xb, yb, H, W, rows, wpb):
    HW = H * W
    C = rows * ROWLEN
    mesh = plsc.VectorSubcoreMesh(core_axis_name="c", subcore_axis_name="s")

    def body(img, xa_h, ya_h, xb_h, yb_h, za_o, zb_o,
             xa_v, ya_v, xb_v, yb_v, ia_v, ib_v, za_v, zb_v, sem):
        c = lax.axis_index("c")
        s = lax.axis_index("s")
        wid = s * NC + c
        boff = (wid // wpb) * HW

        pltpu.sync_copy(xa_h.at[wid], xa_v)
        pltpu.sync_copy(ya_h.at[wid], ya_v)

        def rows_a(j, carry):
            for k in range(ROWLEN // 16):
                sl = pl.ds(j * ROWLEN + k * 16, 16)
                ia_v[sl] = ya_v[sl] * W + xa_v[sl] + boff
            return carry

        lax.fori_loop(0, rows, rows_a, 0)
        da = pltpu.async_copy(img.at[ia_v], za_v, sem)

        pltpu.sync_copy(xb_h.at[wid], xb_v)
        pltpu.sync_copy(yb_h.at[wid], yb_v)

        def rows_b(j, carry):
            for k in range(ROWLEN // 16):
                sl = pl.ds(j * ROWLEN + k * 16, 16)
                ib_v[sl] = yb_v[sl] * W + xb_v[sl] + boff
            return carry

        lax.fori_loop(0, rows, rows_b, 0)
        db = pltpu.async_copy(img.at[ib_v], zb_v, sem)

        da.wait()
        pltpu.sync_copy(za_v, za_o.at[wid])
        db.wait()
        pltpu.sync_copy(zb_v, zb_o.at[wid])

    f = pl.kernel(
        body,
        out_type=(
            jax.ShapeDtypeStruct((NW, C), jnp.float32),
            jax.ShapeDtypeStruct((NW, C), jnp.float32),
        ),
        mesh=mesh,
        scratch_types=[
            pltpu.VMEM((C,), jnp.int32),
            pltpu.VMEM((C,), jnp.int32),
            pltpu.VMEM((C,), jnp.int32),
            pltpu.VMEM((C,), jnp.int32),
            pltpu.VMEM((C,), jnp.int32),
            pltpu.VMEM((C,), jnp.int32),
            pltpu.VMEM((C,), jnp.float32),
            pltpu.VMEM((C,), jnp.float32),
            pltpu.SemaphoreType.DMA,
        ],
    )
    return f(img_flat, xa, ya, xb, yb)


def _tc_loss(za, zb, ordp, n_total):
    def body(za_ref, zb_ref, o_ref, out_ref):
        gt = o_ref[...].astype(jnp.float32) - 1.0
        mask = jnp.abs(gt)
        diff = za_ref[...] - zb_ref[...]
        loss = mask * jnp.log(1.0 + jnp.exp(-gt * diff)) \
            + (1.0 - mask) * diff * diff
        out_ref[0, 0] = jnp.sum(loss) / n_total

    return pl.pallas_call(
        body,
        out_shape=jax.ShapeDtypeStruct((1, 1), jnp.float32),
        out_specs=pl.BlockSpec(memory_space=pltpu.SMEM),
    )(za, zb, ordp)


def kernel(input, x_A, y_A, x_B, y_B, ordinal):
    B, _, H, W = input.shape
    P = x_A.shape[1]
    wpb = NW // B
    rows = -(-P // (wpb * ROWLEN))
    p_pad = rows * ROWLEN * wpb
    pad = p_pad - P

    img = input.reshape(-1)

    def prep(a, cval):
        a = jnp.pad(a, ((0, 0), (0, pad)), constant_values=cval)
        return a.reshape(NW, rows * ROWLEN).astype(jnp.int32)

    xa = prep(x_A, 0)
    ya = prep(y_A, 0)
    xb = prep(x_B, 0)
    yb = prep(y_B, 0)
    op = prep(ordinal, 1)

    za, zb = _sc_gather(img, xa, ya, xb, yb, H, W, rows, wpb)

    R = NW * rows
    loss = _tc_loss(
        za.reshape(R, ROWLEN),
        zb.reshape(R, ROWLEN),
        op.reshape(R, ROWLEN),
        B * P,
    )
    return loss.reshape(1)


# speedup vs baseline: 1.0430x; 1.0430x over previous
import jax
import jax.numpy as jnp
from jax import lax
from jax.experimental import pallas as pl
from jax.experimental.pallas import tpu as pltpu
from jax.experimental.pallas import tpu_sc as plsc

NC, NS = 2, 16
NW = NC * NS
LANES = 16


def _sc_loss_partials(img_flat, xa, ya, xb, yb, ordf, H, W, P, wpb):
    HW = H * W
    own = P // wpb
    C = -(-own // 128) * 128
    for s in range(wpb):
        smod = s * own
        pre_min = max(smod + C - P, 0)
        pre = pre_min + ((smod - pre_min) & 7)
        assert 0 <= smod - pre and smod - pre + C <= P and (smod - pre) % 8 == 0
        assert pre + own <= C
    assert wpb & (wpb - 1) == 0
    wpb_shift = wpb.bit_length() - 1

    mesh = plsc.VectorSubcoreMesh(core_axis_name="c", subcore_axis_name="s")

    def body(img, xa_h, ya_h, xb_h, yb_h, od_h, out_h,
             xa_v, ya_v, xb_v, yb_v, od_v, ia_v, ib_v, za_v, zb_v,
             acc_v, ssem, gsem):
        cc = lax.axis_index("c")
        ss = lax.axis_index("s")
        wid = ss * NC + cc
        b = wid >> wpb_shift
        slot = wid - (b << wpb_shift)
        boff = b * HW
        smod = slot * own
        pre_min = lax.max(smod + (C - P), 0)
        pre = pre_min + ((smod - pre_min) & 7)
        fstart = pl.multiple_of(b * P + smod - pre, 8)

        cp = [
            pltpu.async_copy(xa_h.at[pl.ds(fstart, C)], xa_v, ssem),
            pltpu.async_copy(ya_h.at[pl.ds(fstart, C)], ya_v, ssem),
            pltpu.async_copy(xb_h.at[pl.ds(fstart, C)], xb_v, ssem),
            pltpu.async_copy(yb_h.at[pl.ds(fstart, C)], yb_v, ssem),
            pltpu.async_copy(od_h.at[pl.ds(fstart, C)], od_v, ssem),
        ]
        for d in cp:
            d.wait()

        def rows_a(j, carry):
            for k in range(8):
                sl = pl.ds(j * 128 + k * 16, 16)
                ia_v[sl] = ya_v[sl] * W + xa_v[sl] + boff
            return carry

        lax.fori_loop(0, C // 128, rows_a, 0)
        da = pltpu.async_copy(img.at[ia_v], za_v, gsem)

        def rows_b(j, carry):
            for k in range(8):
                sl = pl.ds(j * 128 + k * 16, 16)
                ib_v[sl] = yb_v[sl] * W + xb_v[sl] + boff
            return carry

        lax.fori_loop(0, C // 128, rows_b, 0)
        db = pltpu.async_copy(img.at[ib_v], zb_v, gsem)
        da.wait()
        db.wait()

        lane = lax.iota(jnp.int32, LANES)
        lo = pre
        hi = pre + own

        def loss_rows(j, acc):
            base = j * 128
            for k in range(8):
                off = base + k * 16
                sl = pl.ds(off, 16)
                gt = od_v[sl].astype(jnp.float32) - 1.0
                diff = za_v[sl] - zb_v[sl]
                q = gt * diff
                u = jnp.exp(-jnp.abs(q))
                t = u / (2.0 + u)
                t2 = t * t
                sp = t * (2.0 + t2 * (2.0 / 3.0 + t2 * (
                    2.0 / 5.0 + t2 * (2.0 / 7.0 + t2 * (2.0 / 9.0)))))
                sp = jnp.maximum(-q, 0.0) + sp
                m = jnp.abs(gt)
                lv = m * sp + (1.0 - m) * (diff * diff)
                li = lane + off
                sel = (li >= lo) & (li < hi)
                acc = acc + jnp.where(sel, lv, 0.0)
            return acc

        acc = lax.fori_loop(0, C // 128, loss_rows,
                            jnp.zeros((LANES,), jnp.float32))
        acc_v[...] = acc
        pltpu.sync_copy(acc_v, out_h.at[pl.ds(wid * LANES, LANES)])

    f = pl.kernel(
        body,
        out_type=jax.ShapeDtypeStruct((NW * LANES,), jnp.float32),
        mesh=mesh,
        scratch_types=[
            pltpu.VMEM((C,), jnp.int32),
            pltpu.VMEM((C,), jnp.int32),
            pltpu.VMEM((C,), jnp.int32),
            pltpu.VMEM((C,), jnp.int32),
            pltpu.VMEM((C,), jnp.int32),
            pltpu.VMEM((C,), jnp.int32),
            pltpu.VMEM((C,), jnp.int32),
            pltpu.VMEM((C,), jnp.float32),
            pltpu.VMEM((C,), jnp.float32),
            pltpu.VMEM((LANES,), jnp.float32),
            pltpu.SemaphoreType.DMA,
            pltpu.SemaphoreType.DMA,
        ],
    )
    return f(img_flat, xa, ya, xb, yb, ordf)


def _tc_reduce(partials, n_total):
    def body(p_ref, out_ref):
        out_ref[0, 0] = jnp.sum(p_ref[...]) / n_total

    return pl.pallas_call(
        body,
        out_shape=jax.ShapeDtypeStruct((1, 1), jnp.float32),
        out_specs=pl.BlockSpec(memory_space=pltpu.SMEM),
    )(partials)


def kernel(input, x_A, y_A, x_B, y_B, ordinal):
    B, _, H, W = input.shape
    P = x_A.shape[1]
    wpb = NW // B

    img = input.reshape(-1)
    flat = lambda a: a.reshape(-1).astype(jnp.int32)

    partials = _sc_loss_partials(
        img, flat(x_A), flat(y_A), flat(x_B), flat(y_B), flat(ordinal),
        H, W, P, wpb)

    loss = _tc_reduce(partials.reshape(4, NW * LANES // 4), B * P)
    return loss.reshape(1)
